# baseline (speedup 1.0000x reference)
import functools

import jax
import jax.numpy as jnp
from jax import lax
from jax.experimental import pallas as pl
from jax.experimental.pallas import tpu as pltpu
from jax.experimental.pallas import tpu_sc as plsc

_BETA = 0.5


def _argmin_body(x_ref, w_ref, xsq_ref, wsq_ref, tok_ref, loss_ref, *, bk: int,
                 prec=None):
    i = pl.program_id(0)
    k_total, d = w_ref.shape
    bm = x_ref.shape[0]
    nk = k_total // bk

    x = x_ref[...]
    x_sq = xsq_ref[...]

    run_min = jnp.full((bm,), jnp.inf, dtype=jnp.float32)
    run_idx = jnp.zeros((bm,), dtype=jnp.int32)
    for kb in range(nk):
        w = w_ref[pl.ds(kb * bk, bk), :]
        wsq = wsq_ref[:, pl.ds(kb * bk, bk)]
        dot = lax.dot_general(x, w, (((1,), (1,)), ((), ())),
                              preferred_element_type=jnp.float32,
                              precision=prec)
        dist = jnp.sqrt(jnp.maximum((x_sq + wsq) - 2.0 * dot, 0.0))
        bmin = jnp.min(dist, axis=1)
        col = lax.broadcasted_iota(jnp.int32, (bm, bk), 1)
        bidx = jnp.min(jnp.where(dist == bmin[:, None], col, k_total), axis=1)
        upd = bmin < run_min
        run_idx = jnp.where(upd, bidx + kb * bk, run_idx)
        run_min = jnp.where(upd, bmin, run_min)

    tok_ref[...] = run_idx
    part = jnp.sum(run_min * run_min)

    @pl.when(i == 0)
    def _first():
        loss_ref[0, 0] = part

    @pl.when(i > 0)
    def _rest():
        loss_ref[0, 0] = loss_ref[0, 0] + part


def _tokens_and_loss(flat_x, w, x_sq, w_sq, bm=256, bk=1024, prec=None):
    m, d = flat_x.shape
    k = w.shape[0]
    grid = (m // bm,)
    return pl.pallas_call(
        functools.partial(_argmin_body, bk=bk, prec=prec),
        grid=grid,
        in_specs=[
            pl.BlockSpec((bm, d), lambda i: (i, 0)),
            pl.BlockSpec((k, d), lambda i: (0, 0)),
            pl.BlockSpec((bm, 1), lambda i: (i, 0)),
            pl.BlockSpec((1, k), lambda i: (0, 0)),
        ],
        out_specs=[
            pl.BlockSpec((bm,), lambda i: (i,)),
            pl.BlockSpec(memory_space=pltpu.SMEM),
        ],
        out_shape=[
            jax.ShapeDtypeStruct((m,), jnp.int32),
            jax.ShapeDtypeStruct((1, 1), jnp.float32),
        ],
    )(flat_x, w, x_sq, w_sq)


def _make_sc_gather(m, k, d):
    info = plsc.get_sparse_core_info()
    nc, ns = info.num_cores, info.num_subcores
    nw = nc * ns
    b_per_w = m // nw
    assert m % nw == 0 and b_per_w % 8 == 0
    ch = 96
    assert b_per_w % ch == 0
    nch = b_per_w // ch
    mesh = plsc.VectorSubcoreMesh(core_axis_name="c", subcore_axis_name="s")

    @functools.partial(
        pl.kernel,
        mesh=mesh,
        out_type=jax.ShapeDtypeStruct((m, d), jnp.float32),
        scratch_types=[
            pltpu.VMEM((ch,), jnp.int32),
            pltpu.VMEM((ch,), jnp.int32),
            pltpu.VMEM((ch, d), jnp.float32),
            pltpu.VMEM((ch, d), jnp.float32),
            pltpu.SemaphoreType.DMA,
            pltpu.SemaphoreType.DMA,
        ],
    )
    def gather(w_hbm, tok_hbm, out_hbm, idx0, idx1, buf0, buf1, sem0, sem1):
        wid = lax.axis_index("s") * nc + lax.axis_index("c")
        base = wid * b_per_w
        idx = (idx0, idx1)
        buf = (buf0, buf1)
        sem = (sem0, sem1)
        handles = [None, None]

        def start(c, slot):
            pltpu.sync_copy(tok_hbm.at[pl.ds(base + c * ch, ch)], idx[slot])
            handles[slot] = pltpu.async_copy(w_hbm.at[idx[slot]], buf[slot], sem[slot])

        start(0, 0)
        for c in range(nch):
            slot = c % 2
            if c + 1 < nch:
                start(c + 1, 1 - slot)
            handles[slot].wait()
            pltpu.sync_copy(buf[slot], out_hbm.at[pl.ds(base + c * ch, ch)])

    return gather


def kernel(x, w):
    b, n, d = x.shape
    k = w.shape[0]
    m = b * n
    flat_x = x.reshape(m, d)
    x_sq = jnp.sum(flat_x * flat_x, axis=1, keepdims=True)
    w_sq = jnp.sum(w * w, axis=1).reshape(1, k)
    tokens_flat, loss_sum = _tokens_and_loss(flat_x, w, x_sq, w_sq)
    quantized = _make_sc_gather(m, k, d)(w, tokens_flat)
    tokens = tokens_flat.reshape(b, n)
    quantized_st = quantized.reshape(b, n, d)
    vq_loss = (1.0 + _BETA) * loss_sum[0, 0] / jnp.float32(m * d)
    return (tokens, quantized_st, vq_loss)


# speedup vs baseline: 1.2549x; 1.2549x over previous
import functools

import jax
import jax.numpy as jnp
from jax import lax
from jax.experimental import pallas as pl
from jax.experimental.pallas import tpu as pltpu
from jax.experimental.pallas import tpu_sc as plsc

_BETA = 0.5


def _argmin_body(x_ref, w2_ref, xsq_ref, wsq_ref, tok_ref, loss_ref,
                 accs_ref, acci_ref, *, bk: int, prec=None):
    i = pl.program_id(0)
    k_total, d = w2_ref.shape
    bm = x_ref.shape[0]
    nk = k_total // bk

    x = x_ref[...]
    x_sq = xsq_ref[...]

    for kb in range(nk):
        w2 = w2_ref[pl.ds(kb * bk, bk), :]
        wsq = wsq_ref[:, pl.ds(kb * bk, bk)]
        dot2 = lax.dot_general(x, w2, (((1,), (1,)), ((), ())),
                               preferred_element_type=jnp.float32,
                               precision=prec)
        s_elem = jnp.sqrt(jnp.maximum((x_sq + wsq) - dot2, 0.0))
        col = lax.broadcasted_iota(jnp.int32, (bm, bk), 1) + (kb * bk)
        if kb == 0:
            accs_ref[...] = s_elem
            acci_ref[...] = col
        else:
            prev = accs_ref[...]
            better = s_elem < prev
            acci_ref[...] = jnp.where(better, col, acci_ref[...])
            accs_ref[...] = jnp.minimum(prev, s_elem)

    acc_s = accs_ref[...]
    s_row = jnp.min(acc_s, axis=1, keepdims=True)
    tok_ref[...] = jnp.min(
        jnp.where(acc_s == s_row, acci_ref[...], k_total), axis=1)

    part = jnp.sum(s_row * s_row)

    @pl.when(i == 0)
    def _first():
        loss_ref[0, 0] = part

    @pl.when(i > 0)
    def _rest():
        loss_ref[0, 0] = loss_ref[0, 0] + part


def _tokens_and_loss(flat_x, w2, x_sq, w_sq, bm=256, bk=1024, prec=None):
    m, d = flat_x.shape
    k = w2.shape[0]
    grid = (m // bm,)
    return pl.pallas_call(
        functools.partial(_argmin_body, bk=bk, prec=prec),
        grid=grid,
        in_specs=[
            pl.BlockSpec((bm, d), lambda i: (i, 0)),
            pl.BlockSpec((k, d), lambda i: (0, 0)),
            pl.BlockSpec((bm, 1), lambda i: (i, 0)),
            pl.BlockSpec((1, k), lambda i: (0, 0)),
        ],
        out_specs=[
            pl.BlockSpec((bm,), lambda i: (i,)),
            pl.BlockSpec(memory_space=pltpu.SMEM),
        ],
        out_shape=[
            jax.ShapeDtypeStruct((m,), jnp.int32),
            jax.ShapeDtypeStruct((1, 1), jnp.float32),
        ],
        scratch_shapes=[pltpu.VMEM((bm, bk), jnp.float32),
                        pltpu.VMEM((bm, bk), jnp.int32)],
    )(flat_x, w2, x_sq, w_sq)


def _make_sc_gather(m, k, d):
    info = plsc.get_sparse_core_info()
    nc, ns = info.num_cores, info.num_subcores
    nw = nc * ns
    b_per_w = m // nw
    assert m % nw == 0 and b_per_w % 8 == 0
    ch = 96
    assert b_per_w % ch == 0
    nch = b_per_w // ch
    mesh = plsc.VectorSubcoreMesh(core_axis_name="c", subcore_axis_name="s")

    @functools.partial(
        pl.kernel,
        mesh=mesh,
        out_type=jax.ShapeDtypeStruct((m, d), jnp.float32),
        scratch_types=[
            pltpu.VMEM((ch,), jnp.int32),
            pltpu.VMEM((ch,), jnp.int32),
            pltpu.VMEM((ch, d), jnp.float32),
            pltpu.VMEM((ch, d), jnp.float32),
            pltpu.SemaphoreType.DMA,
            pltpu.SemaphoreType.DMA,
        ],
    )
    def gather(w_hbm, tok_hbm, out_hbm, idx0, idx1, buf0, buf1, sem0, sem1):
        wid = lax.axis_index("s") * nc + lax.axis_index("c")
        base = wid * b_per_w
        idx = (idx0, idx1)
        buf = (buf0, buf1)
        sem = (sem0, sem1)
        handles = [None, None]

        def start(c, slot):
            pltpu.sync_copy(tok_hbm.at[pl.ds(base + c * ch, ch)], idx[slot])
            handles[slot] = pltpu.async_copy(w_hbm.at[idx[slot]], buf[slot], sem[slot])

        start(0, 0)
        for c in range(nch):
            slot = c % 2
            if c + 1 < nch:
                start(c + 1, 1 - slot)
            handles[slot].wait()
            pltpu.sync_copy(buf[slot], out_hbm.at[pl.ds(base + c * ch, ch)])

    return gather


def kernel(x, w):
    b, n, d = x.shape
    k = w.shape[0]
    m = b * n
    flat_x = x.reshape(m, d)
    x_sq = jnp.sum(flat_x * flat_x, axis=1, keepdims=True)
    w_sq = jnp.sum(w * w, axis=1).reshape(1, k)
    tokens_flat, loss_sum = _tokens_and_loss(flat_x, 2.0 * w, x_sq, w_sq)
    quantized = _make_sc_gather(m, k, d)(w, tokens_flat)
    tokens = tokens_flat.reshape(b, n)
    quantized_st = quantized.reshape(b, n, d)
    vq_loss = (1.0 + _BETA) * loss_sum[0, 0] / jnp.float32(m * d)
    return (tokens, quantized_st, vq_loss)
